# Initial kernel scaffold; baseline (speedup 1.0000x reference)
import jax
import jax.numpy as jnp
from jax.experimental import pallas as pl


def kernel(uid, iid, user_table, item_table, W1, b1, W2, b2, W3, b3, W4, b4, W5, b5):
    raise NotImplementedError("write your pallas kernel here")


# speedup vs baseline: 2.7479x; 2.7479x over previous
import functools

import jax
import jax.numpy as jnp
from jax import lax
from jax.experimental import pallas as pl
from jax.experimental.pallas import tpu as pltpu
from jax.experimental.pallas import tpu_sc as plsc

BATCH = 16384
DIM = 128
NC, NS = 2, 16
NW = NC * NS
BPW = BATCH // NW


def _gather_body(uid_hbm, iid_hbm, utab_hbm, itab_hbm, out_u, out_i,
                 idx_v, rows_v, sem):
    wid = lax.axis_index("s") * NC + lax.axis_index("c")
    base = wid * BPW
    pltpu.sync_copy(uid_hbm.at[pl.ds(base, BPW)], idx_v)
    pltpu.async_copy(utab_hbm.at[idx_v], rows_v, sem).wait()
    pltpu.sync_copy(rows_v, out_u.at[pl.ds(base, BPW)])
    pltpu.sync_copy(iid_hbm.at[pl.ds(base, BPW)], idx_v)
    pltpu.async_copy(itab_hbm.at[idx_v], rows_v, sem).wait()
    pltpu.sync_copy(rows_v, out_i.at[pl.ds(base, BPW)])


_gather = pl.kernel(
    _gather_body,
    out_type=(
        jax.ShapeDtypeStruct((BATCH, DIM), jnp.float32),
        jax.ShapeDtypeStruct((BATCH, DIM), jnp.float32),
    ),
    mesh=plsc.VectorSubcoreMesh(core_axis_name="c", subcore_axis_name="s"),
    scratch_types=[
        pltpu.VMEM((BPW,), jnp.int32),
        pltpu.VMEM((BPW, DIM), jnp.float32),
        pltpu.SemaphoreType.DMA,
    ],
)


def _leaky(x):
    return jnp.where(x >= 0, x, 0.01 * x)


def _mlp_body(u_ref, i_ref, w1a, w1b, b1, w2, b2, w3, b3, w4, b4, w5r, b5,
              out_ref):
    h = jnp.dot(u_ref[...], w1a[...], preferred_element_type=jnp.float32)
    h = h + jnp.dot(i_ref[...], w1b[...], preferred_element_type=jnp.float32)
    h = _leaky(h + b1[...])
    h = _leaky(jnp.dot(h, w2[...], preferred_element_type=jnp.float32) + b2[...])
    h = _leaky(jnp.dot(h, w3[...], preferred_element_type=jnp.float32) + b3[...])
    h = _leaky(jnp.dot(h, w4[...], preferred_element_type=jnp.float32) + b4[...])
    out_ref[...] = jnp.sum(h * w5r[...], axis=1, keepdims=True) + b5[...]


def _mlp(u, i, W1a, W1b, b1, W2, b2, W3, b3, W4, b4, w5r, b5):
    TB = 2048
    grid = (BATCH // TB,)
    full = lambda shape: pl.BlockSpec(shape, lambda g: (0,) * len(shape))
    return pl.pallas_call(
        _mlp_body,
        grid=grid,
        in_specs=[
            pl.BlockSpec((TB, DIM), lambda g: (g, 0)),
            pl.BlockSpec((TB, DIM), lambda g: (g, 0)),
            full((DIM, 64)), full((DIM, 64)), full((1, 64)),
            full((64, 64)), full((1, 64)),
            full((64, 32)), full((1, 32)),
            full((32, 16)), full((1, 16)),
            full((1, 16)), full((1, 1)),
        ],
        out_specs=pl.BlockSpec((TB, 1), lambda g: (g, 0)),
        out_shape=jax.ShapeDtypeStruct((BATCH, 1), jnp.float32),
    )(u, i, W1a, W1b, b1, W2, b2, W3, b3, W4, b4, w5r, b5)


def kernel(uid, iid, user_table, item_table, W1, b1, W2, b2, W3, b3, W4, b4,
           W5, b5):
    out_u, out_i = _gather(uid, iid, user_table, item_table)
    return _mlp(out_u, out_i,
                W1[:DIM], W1[DIM:], b1.reshape(1, 64),
                W2, b2.reshape(1, 64),
                W3, b3.reshape(1, 32),
                W4, b4.reshape(1, 16),
                W5.reshape(1, 16), b5.reshape(1, 1))
